# Initial kernel scaffold; baseline (speedup 1.0000x reference)
import jax
import jax.numpy as jnp
from jax import lax
from jax.experimental import pallas as pl
from jax.experimental.pallas import tpu as pltpu
from jax.experimental.pallas import tpu_sc as plsc

N = 10000
E = 320000
D = 128
NC = 2
NS = 16
NW = NC * NS
EPT = E // NW
B = 80
NB = EPT // B
RPT = N // NS
ZR = 125

_mesh = plsc.VectorSubcoreMesh(
    core_axis_name="c", subcore_axis_name="s", num_cores=NC, num_subcores=NS
)


def _sc_body(with_cnt):
    def body(src_hbm, dst_hbm, x_hbm, agg_out, *rest):
        if with_cnt:
            cnt_out, src_v, dst_v, rows_v, zbuf, cnt_v, agg_sh, gsem = rest
        else:
            src_v, dst_v, rows_v, zbuf, cnt_v, agg_sh, gsem = rest
        c = lax.axis_index("c")
        s = lax.axis_index("s")
        wid = c * NS + s

        pltpu.sync_copy(src_hbm.at[wid], src_v)
        pltpu.sync_copy(dst_hbm.at[wid], dst_v)

        zeros16 = jnp.zeros((16,), jnp.float32)

        def zrow(r, carry):
            for j in range(D // 16):
                zbuf[r, pl.ds(j * 16, 16)] = zeros16
            return carry

        lax.fori_loop(0, ZR, zrow, 0)

        if with_cnt:
            def zcnt(i, carry):
                cnt_v[pl.ds(i * 16, 16)] = zeros16
                return carry

            lax.fori_loop(0, N // 16, zcnt, 0)

        for k in range(RPT // ZR):
            pltpu.sync_copy(zbuf, agg_sh.at[pl.ds(s * RPT + k * ZR, ZR)])
        plsc.subcore_barrier()

        ones16 = jnp.ones((16,), jnp.float32)

        def batch(b, carry):
            pltpu.async_copy(x_hbm.at[src_v.at[b]], rows_v, gsem).wait()
            pltpu.sync_copy(rows_v, agg_sh.at[dst_v.at[b]], add=True)
            if with_cnt:
                for j in range(B // 16):
                    d16 = dst_v[b, pl.ds(j * 16, 16)]
                    plsc.addupdate_scatter(cnt_v, [d16], ones16)
            return carry

        lax.fori_loop(0, NB, batch, 0)
        plsc.subcore_barrier()

        pltpu.sync_copy(
            agg_sh.at[pl.ds(s * RPT, RPT)], agg_out.at[c, pl.ds(s * RPT, RPT)]
        )
        if with_cnt:
            pltpu.sync_copy(cnt_v, cnt_out.at[wid])

    return body


def _make_sc(with_cnt):
    out_type = [jax.ShapeDtypeStruct((NC, N, D), jnp.float32)]
    if with_cnt:
        out_type.append(jax.ShapeDtypeStruct((NW, N), jnp.float32))
    return pl.kernel(
        _sc_body(with_cnt),
        out_type=tuple(out_type),
        mesh=_mesh,
        scratch_types=[
            pltpu.VMEM((NB, B), jnp.int32),
            pltpu.VMEM((NB, B), jnp.int32),
            pltpu.VMEM((B, D), jnp.float32),
            pltpu.VMEM((ZR, D), jnp.float32),
            pltpu.VMEM((N,), jnp.float32),
            pltpu.VMEM_SHARED((N, D), jnp.float32),
            pltpu.SemaphoreType.DMA,
        ],
        name="sc_sage_agg_cnt" if with_cnt else "sc_sage_agg",
    )


_sc_agg_cnt = _make_sc(True)
_sc_agg = _make_sc(False)

R = 2000


def _tc_body(leaky):
    def body(ap_ref, cp_ref, x_ref, wl_ref, wr_ref, b_ref, o_ref):
        p = ap_ref[...]
        agg = p[0] + p[1]
        cnt = jnp.sum(cp_ref[...], axis=0)
        scale = 1.0 / jnp.maximum(cnt, 1.0)
        m = agg * scale[:, None]
        y = jnp.dot(m, wl_ref[...], preferred_element_type=jnp.float32)
        y = y + b_ref[...]
        y = y + jnp.dot(x_ref[...], wr_ref[...], preferred_element_type=jnp.float32)
        if leaky:
            y = jnp.where(y > 0, y, 0.01 * y)
        o_ref[...] = y

    return body


def _tc_layer(agg_parts, cnt_parts, x, wlT, wrT, b2d, leaky):
    return pl.pallas_call(
        _tc_body(leaky),
        grid=(N // R,),
        in_specs=[
            pl.BlockSpec((NC, R, D), lambda i: (0, i, 0)),
            pl.BlockSpec((NW, R), lambda i: (0, i)),
            pl.BlockSpec((R, D), lambda i: (i, 0)),
            pl.BlockSpec((D, D), lambda i: (0, 0)),
            pl.BlockSpec((D, D), lambda i: (0, 0)),
            pl.BlockSpec((1, D), lambda i: (0, 0)),
        ],
        out_specs=pl.BlockSpec((R, D), lambda i: (i, 0)),
        out_shape=jax.ShapeDtypeStruct((N, D), jnp.float32),
        name="tc_sage_layer",
    )(agg_parts, cnt_parts, x, wlT, wrT, b2d)


@jax.jit
def kernel(features, edges, edges2, edge_features, additional_feature,
           W1l, W1r, b1, W2l, W2r, b2):
    src = edges[0].astype(jnp.int32).reshape(NW, NB, B)
    dst = edges[1].astype(jnp.int32).reshape(NW, NB, B)

    agg1, cnt_parts = _sc_agg_cnt(src, dst, features)
    h = _tc_layer(agg1, cnt_parts, features, W1l.T, W1r.T,
                  b1.reshape(1, D), leaky=True)
    (agg2,) = _sc_agg(src, dst, h)
    out = _tc_layer(agg2, cnt_parts, h, W2l.T, W2r.T,
                    b2.reshape(1, D), leaky=False)
    return out


# speedup vs baseline: 7.9240x; 7.9240x over previous
import jax
import jax.numpy as jnp
from jax import lax
from jax.experimental import pallas as pl
from jax.experimental.pallas import tpu as pltpu
from jax.experimental.pallas import tpu_sc as plsc

N = 10000
E = 320000
D = 128
NC = 2
NS = 16
NW = NC * NS
EPT = E // NW
B = 80
NB = EPT // B
NG = 5
CB = NB // NG
RPT = 624
REM = N - RPT * NS
ZR = 48

_mesh = plsc.VectorSubcoreMesh(
    core_axis_name="c", subcore_axis_name="s", num_cores=NC, num_subcores=NS
)


def _sc_body(with_cnt):
    def body(src_hbm, dst_hbm, x_hbm, agg_out, *rest):
        if with_cnt:
            cnt_out, src_v, dst_v, rows_v, zbuf, cnt_v, agg_sh, gsem = rest
        else:
            cnt_v = None
            src_v, dst_v, rows_v, zbuf, agg_sh, gsem = rest
        c = lax.axis_index("c")
        s = lax.axis_index("s")
        wid = c * NS + s

        zeros16 = jnp.zeros((16,), jnp.float32)

        def zrow(r, carry):
            for j in range(D // 16):
                zbuf[r, pl.ds(j * 16, 16)] = zeros16
            return carry

        lax.fori_loop(0, ZR, zrow, 0)

        if with_cnt:
            def zcnt(i, carry):
                cnt_v[pl.ds(i * 16, 16)] = zeros16
                return carry

            lax.fori_loop(0, N // 16, zcnt, 0)

        for k in range(RPT // ZR):
            pltpu.sync_copy(zbuf, agg_sh.at[pl.ds(s * RPT + k * ZR, ZR)])

        @pl.when(s == NS - 1)
        def _zero_rem():
            pltpu.sync_copy(
                zbuf.at[pl.ds(0, REM)], agg_sh.at[pl.ds(RPT * NS, REM)]
            )

        plsc.subcore_barrier()

        ones16 = jnp.ones((16,), jnp.float32)

        def chunk(g, carry):
            pltpu.sync_copy(src_hbm.at[wid, g], src_v)
            pltpu.sync_copy(dst_hbm.at[wid, g], dst_v)

            def batch(b, carry2):
                pltpu.async_copy(x_hbm.at[src_v.at[b]], rows_v, gsem).wait()
                pltpu.sync_copy(rows_v, agg_sh.at[dst_v.at[b]], add=True)
                if with_cnt:
                    for j in range(B // 16):
                        d16 = dst_v[b, pl.ds(j * 16, 16)]
                        plsc.addupdate_scatter(cnt_v, [d16], ones16)
                return carry2

            lax.fori_loop(0, CB, batch, 0)
            return carry

        lax.fori_loop(0, NG, chunk, 0)
        plsc.subcore_barrier()

        pltpu.sync_copy(
            agg_sh.at[pl.ds(s * RPT, RPT)], agg_out.at[c, pl.ds(s * RPT, RPT)]
        )

        @pl.when(s == NS - 1)
        def _copy_rem():
            pltpu.sync_copy(
                agg_sh.at[pl.ds(RPT * NS, REM)],
                agg_out.at[c, pl.ds(RPT * NS, REM)],
            )

        if with_cnt:
            pltpu.sync_copy(cnt_v, cnt_out.at[pl.ds(wid * N, N)])

    return body


def _make_sc(with_cnt):
    out_type = [jax.ShapeDtypeStruct((NC, N, D), jnp.float32)]
    if with_cnt:
        out_type.append(jax.ShapeDtypeStruct((NW * N,), jnp.float32))
    return pl.kernel(
        _sc_body(with_cnt),
        out_type=tuple(out_type),
        mesh=_mesh,
        scratch_types=(
            [
                pltpu.VMEM((CB, B), jnp.int32),
                pltpu.VMEM((CB, B), jnp.int32),
                pltpu.VMEM((B, D), jnp.float32),
                pltpu.VMEM((ZR, D), jnp.float32),
            ]
            + ([pltpu.VMEM((N,), jnp.float32)] if with_cnt else [])
            + [
                pltpu.VMEM_SHARED((N, D), jnp.float32),
                pltpu.SemaphoreType.DMA,
            ]
        ),
        name="sc_sage_agg_cnt" if with_cnt else "sc_sage_agg",
        compiler_params=pltpu.CompilerParams(needs_layout_passes=False),
    )


_sc_agg_cnt = _make_sc(True)
_sc_agg = _make_sc(False)

R = 2000


def _tc_body(leaky):
    def body(ap_ref, cp_ref, x_ref, wl_ref, wr_ref, b_ref, o_ref):
        p = ap_ref[...]
        agg = p[0] + p[1]
        cnt = jnp.sum(cp_ref[...], axis=1)
        scale = 1.0 / jnp.maximum(cnt, 1.0)
        m = agg * scale[:, None]
        y = jnp.dot(m, wl_ref[...], preferred_element_type=jnp.float32)
        y = y + b_ref[...]
        y = y + jnp.dot(x_ref[...], wr_ref[...], preferred_element_type=jnp.float32)
        if leaky:
            y = jnp.where(y > 0, y, 0.01 * y)
        o_ref[...] = y

    return body


def _tc_layer(agg_parts, cnt_parts, x, wlT, wrT, b2d, leaky):
    return pl.pallas_call(
        _tc_body(leaky),
        grid=(N // R,),
        in_specs=[
            pl.BlockSpec((NC, R, D), lambda i: (0, i, 0)),
            pl.BlockSpec((R, NW), lambda i: (i, 0)),
            pl.BlockSpec((R, D), lambda i: (i, 0)),
            pl.BlockSpec((D, D), lambda i: (0, 0)),
            pl.BlockSpec((D, D), lambda i: (0, 0)),
            pl.BlockSpec((1, D), lambda i: (0, 0)),
        ],
        out_specs=pl.BlockSpec((R, D), lambda i: (i, 0)),
        out_shape=jax.ShapeDtypeStruct((N, D), jnp.float32),
        name="tc_sage_layer",
    )(agg_parts, cnt_parts, x, wlT, wrT, b2d)


@jax.jit
def kernel(features, edges, edges2, edge_features, additional_feature,
           W1l, W1r, b1, W2l, W2r, b2):
    src = edges[0].astype(jnp.int32).reshape(NW, NG, CB, B)
    dst = edges[1].astype(jnp.int32).reshape(NW, NG, CB, B)

    agg1, cnt_flat = _sc_agg_cnt(src, dst, features)
    cnt_parts = cnt_flat.reshape(NW, N).T
    h = _tc_layer(agg1, cnt_parts, features, W1l.T, W1r.T,
                  b1.reshape(1, D), leaky=True)
    (agg2,) = _sc_agg(src, dst, h)
    out = _tc_layer(agg2, cnt_parts, h, W2l.T, W2r.T,
                    b2.reshape(1, D), leaky=False)
    return out


# speedup vs baseline: 12.1520x; 1.5336x over previous
import jax
import jax.numpy as jnp
from jax import lax
from jax.experimental import pallas as pl
from jax.experimental.pallas import tpu as pltpu
from jax.experimental.pallas import tpu_sc as plsc

N = 10000
E = 320000
D = 128
NC = 2
NS = 16
NW = NC * NS
EPT = E // NW
B = 80
NB = EPT // B
NG = 5
CB = NB // NG
RPT = 624
REM = N - RPT * NS
ZR = 48

_mesh = plsc.VectorSubcoreMesh(
    core_axis_name="c", subcore_axis_name="s", num_cores=NC, num_subcores=NS
)


def _sc_body(with_cnt):
    def body(src_hbm, dst_hbm, x_hbm, agg_out, *rest):
        if with_cnt:
            (cnt_out, src_v, dst_v, rows_a, rows_b, zbuf, cnt_v, agg_sh,
             gsem_a, gsem_b) = rest
        else:
            cnt_v = None
            src_v, dst_v, rows_a, rows_b, zbuf, agg_sh, gsem_a, gsem_b = rest
        c = lax.axis_index("c")
        s = lax.axis_index("s")
        wid = c * NS + s

        zeros16 = jnp.zeros((16,), jnp.float32)

        def zrow(r, carry):
            for j in range(D // 16):
                zbuf[r, pl.ds(j * 16, 16)] = zeros16
            return carry

        lax.fori_loop(0, ZR, zrow, 0)

        if with_cnt:
            def zcnt(i, carry):
                cnt_v[pl.ds(i * 16, 16)] = zeros16
                return carry

            lax.fori_loop(0, N // 16, zcnt, 0)

        for k in range(RPT // ZR):
            pltpu.sync_copy(zbuf, agg_sh.at[pl.ds(s * RPT + k * ZR, ZR)])

        @pl.when(s == NS - 1)
        def _zero_rem():
            pltpu.sync_copy(
                zbuf.at[pl.ds(0, REM)], agg_sh.at[pl.ds(RPT * NS, REM)]
            )

        plsc.subcore_barrier()

        ones16 = jnp.ones((16,), jnp.float32)
        rows = (rows_a, rows_b)
        gsems = (gsem_a, gsem_b)

        def start_gather(b, buf_i):
            pltpu.async_copy(x_hbm.at[src_v.at[b]], rows[buf_i], gsems[buf_i])

        def finish(b, buf_i):
            pltpu.make_async_copy(
                x_hbm.at[src_v.at[b]], rows[buf_i], gsems[buf_i]
            ).wait()
            pltpu.sync_copy(rows[buf_i], agg_sh.at[dst_v.at[b]], add=True)
            if with_cnt:
                for j in range(B // 16):
                    d16 = dst_v[b, pl.ds(j * 16, 16)]
                    plsc.addupdate_scatter(cnt_v, [d16], ones16)

        def chunk(g, carry):
            pltpu.sync_copy(src_hbm.at[wid, g], src_v)
            pltpu.sync_copy(dst_hbm.at[wid, g], dst_v)
            start_gather(0, 0)

            def pair(p, carry2):
                start_gather(2 * p + 1, 1)
                finish(2 * p, 0)
                start_gather(2 * p + 2, 0)
                finish(2 * p + 1, 1)
                return carry2

            lax.fori_loop(0, (CB - 1) // 2, pair, 0)
            finish(CB - 1, 0)
            return carry

        lax.fori_loop(0, NG, chunk, 0)
        plsc.subcore_barrier()

        pltpu.sync_copy(
            agg_sh.at[pl.ds(s * RPT, RPT)], agg_out.at[c, pl.ds(s * RPT, RPT)]
        )

        @pl.when(s == NS - 1)
        def _copy_rem():
            pltpu.sync_copy(
                agg_sh.at[pl.ds(RPT * NS, REM)],
                agg_out.at[c, pl.ds(RPT * NS, REM)],
            )

        if with_cnt:
            pltpu.sync_copy(cnt_v, cnt_out.at[pl.ds(wid * N, N)])

    return body


def _make_sc(with_cnt):
    out_type = [jax.ShapeDtypeStruct((NC, N, D), jnp.float32)]
    if with_cnt:
        out_type.append(jax.ShapeDtypeStruct((NW * N,), jnp.float32))
    return pl.kernel(
        _sc_body(with_cnt),
        out_type=tuple(out_type),
        mesh=_mesh,
        scratch_types=(
            [
                pltpu.VMEM((CB, B), jnp.int32),
                pltpu.VMEM((CB, B), jnp.int32),
                pltpu.VMEM((B, D), jnp.float32),
                pltpu.VMEM((B, D), jnp.float32),
                pltpu.VMEM((ZR, D), jnp.float32),
            ]
            + ([pltpu.VMEM((N,), jnp.float32)] if with_cnt else [])
            + [
                pltpu.VMEM_SHARED((N, D), jnp.float32),
                pltpu.SemaphoreType.DMA,
                pltpu.SemaphoreType.DMA,
            ]
        ),
        name="sc_sage_agg_cnt" if with_cnt else "sc_sage_agg",
        compiler_params=pltpu.CompilerParams(needs_layout_passes=False),
    )


_sc_agg_cnt = _make_sc(True)
_sc_agg = _make_sc(False)

R = 2000


def _tc_body(leaky):
    def body(ap_ref, cp_ref, x_ref, wl_ref, wr_ref, b_ref, o_ref):
        p = ap_ref[...]
        agg = p[0] + p[1]
        cnt = jnp.sum(cp_ref[...], axis=1)
        scale = 1.0 / jnp.maximum(cnt, 1.0)
        m = agg * scale[:, None]
        y = jnp.dot(m, wl_ref[...], preferred_element_type=jnp.float32)
        y = y + b_ref[...]
        y = y + jnp.dot(x_ref[...], wr_ref[...], preferred_element_type=jnp.float32)
        if leaky:
            y = jnp.where(y > 0, y, 0.01 * y)
        o_ref[...] = y

    return body


def _tc_layer(agg_parts, cnt_parts, x, wlT, wrT, b2d, leaky):
    return pl.pallas_call(
        _tc_body(leaky),
        grid=(N // R,),
        in_specs=[
            pl.BlockSpec((NC, R, D), lambda i: (0, i, 0)),
            pl.BlockSpec((R, NW), lambda i: (i, 0)),
            pl.BlockSpec((R, D), lambda i: (i, 0)),
            pl.BlockSpec((D, D), lambda i: (0, 0)),
            pl.BlockSpec((D, D), lambda i: (0, 0)),
            pl.BlockSpec((1, D), lambda i: (0, 0)),
        ],
        out_specs=pl.BlockSpec((R, D), lambda i: (i, 0)),
        out_shape=jax.ShapeDtypeStruct((N, D), jnp.float32),
        name="tc_sage_layer",
    )(agg_parts, cnt_parts, x, wlT, wrT, b2d)


@jax.jit
def kernel(features, edges, edges2, edge_features, additional_feature,
           W1l, W1r, b1, W2l, W2r, b2):
    src = edges[0].astype(jnp.int32).reshape(NW, NG, CB, B)
    dst = edges[1].astype(jnp.int32).reshape(NW, NG, CB, B)

    agg1, cnt_flat = _sc_agg_cnt(src, dst, features)
    cnt_parts = cnt_flat.reshape(NW, N).T
    h = _tc_layer(agg1, cnt_parts, features, W1l.T, W1r.T,
                  b1.reshape(1, D), leaky=True)
    (agg2,) = _sc_agg(src, dst, h)
    out = _tc_layer(agg2, cnt_parts, h, W2l.T, W2r.T,
                    b2.reshape(1, D), leaky=False)
    return out


# speedup vs baseline: 13.7048x; 1.1278x over previous
import jax
import jax.numpy as jnp
from jax import lax
from jax.experimental import pallas as pl
from jax.experimental.pallas import tpu as pltpu
from jax.experimental.pallas import tpu_sc as plsc

N = 10000
E = 320000
D = 128
NC = 2
NS = 16
NW = NC * NS
EPT = E // NW
B = 80
NB = EPT // B
NG = 5
CB = NB // NG
RPT = 624
REM = N - RPT * NS
ZR = 48

_mesh = plsc.VectorSubcoreMesh(
    core_axis_name="c", subcore_axis_name="s", num_cores=NC, num_subcores=NS
)


def _sc_body(with_cnt):
    def body(src_hbm, dst_hbm, x_hbm, agg_out, *rest):
        if with_cnt:
            (cnt_out, src_v, dst_v, rows_a, rows_b, rows_c, cnt_v, agg_sh,
             gsem_a, gsem_b, gsem_c, ssem_a, ssem_b, ssem_c) = rest
        else:
            cnt_v = None
            (src_v, dst_v, rows_a, rows_b, rows_c, agg_sh,
             gsem_a, gsem_b, gsem_c, ssem_a, ssem_b, ssem_c) = rest
        c = lax.axis_index("c")
        s = lax.axis_index("s")
        wid = c * NS + s

        zeros16 = jnp.zeros((16,), jnp.float32)

        def zrow(r, carry):
            for j in range(D // 16):
                rows_a[r, pl.ds(j * 16, 16)] = zeros16
            return carry

        lax.fori_loop(0, ZR, zrow, 0)

        if with_cnt:
            def zcnt(i, carry):
                cnt_v[pl.ds(i * 16, 16)] = zeros16
                return carry

            lax.fori_loop(0, N // 16, zcnt, 0)

        zsrc = rows_a.at[pl.ds(0, ZR)]
        for k in range(RPT // ZR):
            pltpu.sync_copy(zsrc, agg_sh.at[pl.ds(s * RPT + k * ZR, ZR)])

        @pl.when(s == NS - 1)
        def _zero_rem():
            pltpu.sync_copy(
                rows_a.at[pl.ds(0, REM)], agg_sh.at[pl.ds(RPT * NS, REM)]
            )

        plsc.subcore_barrier()

        ones16 = jnp.ones((16,), jnp.float32)
        rows = (rows_a, rows_b, rows_c)
        gsems = (gsem_a, gsem_b, gsem_c)
        ssems = (ssem_a, ssem_b, ssem_c)

        def start_g(b, k):
            pltpu.async_copy(x_hbm.at[src_v.at[b]], rows[k], gsems[k])

        def wait_g(b, k):
            pltpu.make_async_copy(
                x_hbm.at[src_v.at[b]], rows[k], gsems[k]
            ).wait()

        def start_s(b, k):
            pltpu.async_copy(
                rows[k], agg_sh.at[dst_v.at[b]], ssems[k], add=True
            )

        def wait_s(b, k):
            pltpu.make_async_copy(
                rows[k], agg_sh.at[dst_v.at[b]], ssems[k]
            ).wait()

        def cnt_add(b):
            if with_cnt:
                for j in range(B // 16):
                    d16 = dst_v[b, pl.ds(j * 16, 16)]
                    plsc.addupdate_scatter(cnt_v, [d16], ones16)

        def chunk(g, carry):
            pltpu.sync_copy(src_hbm.at[wid, g], src_v)
            pltpu.sync_copy(dst_hbm.at[wid, g], dst_v)
            start_g(0, 0)
            start_g(1, 1)

            def group(q, carry2):
                for r in range(3):
                    i = 3 * q + r
                    wait_g(i, r)
                    start_s(i, r)
                    cnt_add(i)
                    prev = (r + 2) % 3
                    if r == 0:
                        @pl.when(q > 0)
                        def _ws():
                            wait_s(3 * q - 1, prev)
                    else:
                        wait_s(i - 1, prev)

                    @pl.when(i + 2 < CB)
                    def _sg():
                        start_g(i + 2, prev)
                return carry2

            lax.fori_loop(0, (CB - 1) // 3, group, 0)
            last = CB - 1
            wait_g(last, 0)
            start_s(last, 0)
            cnt_add(last)
            wait_s(last - 1, 2)
            wait_s(last, 0)
            return carry

        lax.fori_loop(0, NG, chunk, 0)
        plsc.subcore_barrier()

        pltpu.sync_copy(
            agg_sh.at[pl.ds(s * RPT, RPT)], agg_out.at[c, pl.ds(s * RPT, RPT)]
        )

        @pl.when(s == NS - 1)
        def _copy_rem():
            pltpu.sync_copy(
                agg_sh.at[pl.ds(RPT * NS, REM)],
                agg_out.at[c, pl.ds(RPT * NS, REM)],
            )

        if with_cnt:
            pltpu.sync_copy(cnt_v, cnt_out.at[pl.ds(wid * N, N)])

    return body


def _make_sc(with_cnt):
    out_type = [jax.ShapeDtypeStruct((NC, N, D), jnp.float32)]
    if with_cnt:
        out_type.append(jax.ShapeDtypeStruct((NW * N,), jnp.float32))
    return pl.kernel(
        _sc_body(with_cnt),
        out_type=tuple(out_type),
        mesh=_mesh,
        scratch_types=(
            [
                pltpu.VMEM((CB, B), jnp.int32),
                pltpu.VMEM((CB, B), jnp.int32),
                pltpu.VMEM((B, D), jnp.float32),
                pltpu.VMEM((B, D), jnp.float32),
                pltpu.VMEM((B, D), jnp.float32),
            ]
            + ([pltpu.VMEM((N,), jnp.float32)] if with_cnt else [])
            + [
                pltpu.VMEM_SHARED((N, D), jnp.float32),
            ]
            + [pltpu.SemaphoreType.DMA] * 6
        ),
        name="sc_sage_agg_cnt" if with_cnt else "sc_sage_agg",
        compiler_params=pltpu.CompilerParams(needs_layout_passes=False),
    )


_sc_agg_cnt = _make_sc(True)
_sc_agg = _make_sc(False)

R = 2000


def _tc_body(leaky):
    def body(ap_ref, cp_ref, x_ref, wl_ref, wr_ref, b_ref, o_ref):
        p = ap_ref[...]
        agg = p[0] + p[1]
        cnt = jnp.sum(cp_ref[...], axis=1)
        scale = 1.0 / jnp.maximum(cnt, 1.0)
        m = agg * scale[:, None]
        y = jnp.dot(m, wl_ref[...], preferred_element_type=jnp.float32)
        y = y + b_ref[...]
        y = y + jnp.dot(x_ref[...], wr_ref[...], preferred_element_type=jnp.float32)
        if leaky:
            y = jnp.where(y > 0, y, 0.01 * y)
        o_ref[...] = y

    return body


def _tc_layer(agg_parts, cnt_parts, x, wlT, wrT, b2d, leaky):
    return pl.pallas_call(
        _tc_body(leaky),
        grid=(N // R,),
        in_specs=[
            pl.BlockSpec((NC, R, D), lambda i: (0, i, 0)),
            pl.BlockSpec((R, NW), lambda i: (i, 0)),
            pl.BlockSpec((R, D), lambda i: (i, 0)),
            pl.BlockSpec((D, D), lambda i: (0, 0)),
            pl.BlockSpec((D, D), lambda i: (0, 0)),
            pl.BlockSpec((1, D), lambda i: (0, 0)),
        ],
        out_specs=pl.BlockSpec((R, D), lambda i: (i, 0)),
        out_shape=jax.ShapeDtypeStruct((N, D), jnp.float32),
        name="tc_sage_layer",
    )(agg_parts, cnt_parts, x, wlT, wrT, b2d)


@jax.jit
def kernel(features, edges, edges2, edge_features, additional_feature,
           W1l, W1r, b1, W2l, W2r, b2):
    src = edges[0].astype(jnp.int32).reshape(NW, NG, CB, B)
    dst = edges[1].astype(jnp.int32).reshape(NW, NG, CB, B)

    agg1, cnt_flat = _sc_agg_cnt(src, dst, features)
    cnt_parts = cnt_flat.reshape(NW, N).T
    h = _tc_layer(agg1, cnt_parts, features, W1l.T, W1r.T,
                  b1.reshape(1, D), leaky=True)
    (agg2,) = _sc_agg(src, dst, h)
    out = _tc_layer(agg2, cnt_parts, h, W2l.T, W2r.T,
                    b2.reshape(1, D), leaky=False)
    return out
